# baseline (speedup 1.0000x reference)
import functools

import jax
import jax.numpy as jnp
from jax.experimental import pallas as pl
from jax.experimental.pallas import tpu as pltpu
from jax.experimental.pallas import tpu_sc as plsc

B = 4096
L = 50
D = 64
W = 128


def kernel(user_id, items_ids, user_table, item_table):
    n_items = B * L
    user_idx = user_id.reshape(1, B).astype(jnp.int32)
    item_idx = items_ids.reshape(1, n_items).astype(jnp.int32)

    mesh = plsc.VectorSubcoreMesh(
        core_axis_name="core", subcore_axis_name="subcore"
    )

    @functools.partial(
        pl.kernel,
        out_type=(
            jax.ShapeDtypeStruct((B, D), jnp.float32),
            jax.ShapeDtypeStruct((n_items, D), jnp.float32),
        ),
        mesh=mesh,
        compiler_params=pltpu.CompilerParams(use_tc_tiling_on_sc=False),
    )
    def run(ut_hbm, it_hbm, uidx_hbm, iidx_hbm, uo_hbm, io_hbm):
        def gather_body(table_hbm):
            def body(i_vmem, o_vmem):
                pltpu.sync_copy(table_hbm.at[i_vmem.at[0]], o_vmem)

            return body

        pltpu.emit_pipeline(
            gather_body(ut_hbm),
            grid=(B // W,),
            in_specs=[pl.BlockSpec((1, W), index_map=lambda i: (0, i))],
            out_specs=[pl.BlockSpec((W, D), index_map=lambda i: (i, 0))],
            core_axis_name=("core", "subcore"),
            dimension_semantics=(pltpu.PARALLEL,),
        )(uidx_hbm, uo_hbm)

        pltpu.emit_pipeline(
            gather_body(it_hbm),
            grid=(n_items // W,),
            in_specs=[pl.BlockSpec((1, W), index_map=lambda i: (0, i))],
            out_specs=[pl.BlockSpec((W, D), index_map=lambda i: (i, 0))],
            core_axis_name=("core", "subcore"),
            dimension_semantics=(pltpu.PARALLEL,),
        )(iidx_hbm, io_hbm)

    user_eb, item_flat = run(user_table, item_table, user_idx, item_idx)
    return user_eb, item_flat.reshape(B, L, D)


# speedup vs baseline: 1.0212x; 1.0212x over previous
import functools

import jax
import jax.numpy as jnp
from jax import lax
from jax.experimental import pallas as pl
from jax.experimental.pallas import tpu as pltpu
from jax.experimental.pallas import tpu_sc as plsc

B = 4096
L = 50
D = 64
NC = 2
NS = 16
NW = NC * NS

N_ITEMS = B * L
IPW = N_ITEMS // NW
UPW = B // NW

CH = 128
NCH = IPW // CH
K = 5
NBUF = 2 * K
STEP = NBUF


def kernel(user_id, items_ids, user_table, item_table):
    user_idx = user_id.reshape(B).astype(jnp.int32)
    item_idx = items_ids.reshape(N_ITEMS).astype(jnp.int32)

    mesh = plsc.VectorSubcoreMesh(
        core_axis_name="core", subcore_axis_name="subcore"
    )

    @functools.partial(
        pl.kernel,
        out_type=(
            jax.ShapeDtypeStruct((B, D), jnp.float32),
            jax.ShapeDtypeStruct((N_ITEMS, D), jnp.float32),
        ),
        mesh=mesh,
        scratch_types=[
            pltpu.VMEM((IPW,), jnp.int32),
            pltpu.VMEM((NBUF * CH, D), jnp.float32),
            pltpu.VMEM((UPW,), jnp.int32),
            pltpu.VMEM((UPW, D), jnp.float32),
            pltpu.SemaphoreType.DMA((NBUF,)),
            pltpu.SemaphoreType.DMA((NBUF,)),
            pltpu.SemaphoreType.DMA,
        ],
        compiler_params=pltpu.CompilerParams(use_tc_tiling_on_sc=False),
    )
    def run(ut_hbm, it_hbm, uidx_hbm, iidx_hbm, uo_hbm, io_hbm,
            idx_v, bufs, uidx_v, ubuf, gsem, ssem, usem):
        c = lax.axis_index("core")
        s = lax.axis_index("subcore")
        wid = s * NC + c
        ibase = wid * IPW
        ubase = wid * UPW

        pltpu.sync_copy(iidx_hbm.at[pl.ds(ibase, IPW)], idx_v)
        pltpu.sync_copy(uidx_hbm.at[pl.ds(ubase, UPW)], uidx_v)

        pltpu.async_copy(ut_hbm.at[uidx_v], ubuf, usem)

        def gather_chunk(j, b):
            pltpu.async_copy(
                it_hbm.at[idx_v.at[pl.ds(j * CH, CH)]],
                bufs.at[pl.ds(b * CH, CH)],
                gsem.at[b],
            )

        def store_chunk(j, b):
            pltpu.async_copy(
                bufs.at[pl.ds(b * CH, CH)],
                io_hbm.at[pl.ds(ibase + j * CH, CH)],
                ssem.at[b],
            )

        for b in range(K):
            gather_chunk(b, b)

        @pl.loop(0, NCH, step=STEP)
        def _(j0):
            for b in range(NBUF):
                j = j0 + b
                pltpu.make_async_copy(
                    it_hbm.at[idx_v.at[pl.ds(j * CH, CH)]],
                    bufs.at[pl.ds(b * CH, CH)],
                    gsem.at[b],
                ).wait()
                store_chunk(j, b)
                bn = (b + K) % NBUF
                jn = j + K

                @pl.when(jn < NCH)
                def _():
                    @pl.when(j >= K)
                    def _():
                        pltpu.make_async_copy(
                            bufs.at[pl.ds(bn * CH, CH)],
                            io_hbm.at[pl.ds(ibase + (jn - NBUF) * CH, CH)],
                            ssem.at[bn],
                        ).wait()

                    gather_chunk(jn, bn)

        for b in range(NBUF):
            j = NCH - NBUF + b
            pltpu.make_async_copy(
                bufs.at[pl.ds(b * CH, CH)],
                io_hbm.at[pl.ds(ibase + j * CH, CH)],
                ssem.at[b % NBUF],
            ).wait()

        pltpu.make_async_copy(ut_hbm.at[uidx_v], ubuf, usem).wait()
        pltpu.sync_copy(ubuf, uo_hbm.at[pl.ds(ubase, UPW)])

    user_eb, item_flat = run(user_table, item_table, user_idx, item_idx)
    return user_eb, item_flat.reshape(B, L, D)


# speedup vs baseline: 1.0322x; 1.0107x over previous
import functools

import jax
import jax.numpy as jnp
from jax import lax
from jax.experimental import pallas as pl
from jax.experimental.pallas import tpu as pltpu
from jax.experimental.pallas import tpu_sc as plsc

B = 4096
L = 50
D = 64
NC = 2
NS = 16
NW = NC * NS

N_ITEMS = B * L
IPW = N_ITEMS // NW
UPW = B // NW

CH = 128
NCH = IPW // CH
K = 5
NBUF = 2 * K
STEP = NBUF


def kernel(user_id, items_ids, user_table, item_table):
    user_idx = user_id.reshape(B).astype(jnp.int32)
    item_idx = items_ids.reshape(N_ITEMS).astype(jnp.int32)

    mesh = plsc.VectorSubcoreMesh(
        core_axis_name="core", subcore_axis_name="subcore"
    )

    @functools.partial(
        pl.kernel,
        out_type=jax.ShapeDtypeStruct((N_ITEMS, D), jnp.float32),
        mesh=mesh,
        scratch_types=[
            pltpu.VMEM((IPW,), jnp.int32),
            pltpu.VMEM((NBUF * CH, D), jnp.float32),
            pltpu.SemaphoreType.DMA((NBUF,)),
            pltpu.SemaphoreType.DMA((NBUF,)),
        ],
        compiler_params=pltpu.CompilerParams(use_tc_tiling_on_sc=False),
    )
    def run_items(it_hbm, iidx_hbm, io_hbm, idx_v, bufs, gsem, ssem):
        c = lax.axis_index("core")
        s = lax.axis_index("subcore")
        wid = s * NC + c
        ibase = wid * IPW

        pltpu.sync_copy(iidx_hbm.at[pl.ds(ibase, IPW)], idx_v)

        def gather_chunk(j, b):
            pltpu.async_copy(
                it_hbm.at[idx_v.at[pl.ds(j * CH, CH)]],
                bufs.at[pl.ds(b * CH, CH)],
                gsem.at[b],
            )

        def store_chunk(j, b):
            pltpu.async_copy(
                bufs.at[pl.ds(b * CH, CH)],
                io_hbm.at[pl.ds(ibase + j * CH, CH)],
                ssem.at[b],
            )

        for b in range(K):
            gather_chunk(b, b)

        @pl.loop(0, NCH, step=STEP)
        def _(j0):
            for b in range(NBUF):
                j = j0 + b
                pltpu.make_async_copy(
                    it_hbm.at[idx_v.at[pl.ds(j * CH, CH)]],
                    bufs.at[pl.ds(b * CH, CH)],
                    gsem.at[b],
                ).wait()
                store_chunk(j, b)
                bn = (b + K) % NBUF
                jn = j + K

                @pl.when(jn < NCH)
                def _():
                    @pl.when(j >= K)
                    def _():
                        pltpu.make_async_copy(
                            bufs.at[pl.ds(bn * CH, CH)],
                            io_hbm.at[pl.ds(ibase + (jn - NBUF) * CH, CH)],
                            ssem.at[bn],
                        ).wait()

                    gather_chunk(jn, bn)

        for b in range(NBUF):
            j = NCH - NBUF + b
            pltpu.make_async_copy(
                bufs.at[pl.ds(b * CH, CH)],
                io_hbm.at[pl.ds(ibase + j * CH, CH)],
                ssem.at[b % NBUF],
            ).wait()

    @functools.partial(
        pl.kernel,
        out_type=jax.ShapeDtypeStruct((B, D), jnp.float32),
        mesh=mesh,
        scratch_types=[
            pltpu.VMEM((UPW,), jnp.int32),
            pltpu.VMEM((UPW, D), jnp.float32),
            pltpu.SemaphoreType.DMA,
        ],
        compiler_params=pltpu.CompilerParams(use_tc_tiling_on_sc=False),
    )
    def run_user(ut_hbm, uidx_hbm, uo_hbm, uidx_v, ubuf, usem):
        c = lax.axis_index("core")
        s = lax.axis_index("subcore")
        wid = s * NC + c
        ubase = wid * UPW
        pltpu.sync_copy(uidx_hbm.at[pl.ds(ubase, UPW)], uidx_v)
        pltpu.async_copy(ut_hbm.at[uidx_v], ubuf, usem).wait()
        pltpu.sync_copy(ubuf, uo_hbm.at[pl.ds(ubase, UPW)])

    item_flat = run_items(item_table, item_idx)
    user_eb = run_user(user_table, user_idx)
    return user_eb, item_flat.reshape(B, L, D)
